# Initial kernel scaffold; baseline (speedup 1.0000x reference)
import jax
import jax.numpy as jnp
from jax.experimental import pallas as pl


def kernel(image, feature, sp, num):
    raise NotImplementedError("write your pallas kernel here")


# speedup vs baseline: 17.7817x; 17.7817x over previous
import numpy as np

import jax
import jax.numpy as jnp
from jax.experimental import pallas as pl
from jax.experimental.pallas import tpu as pltpu

_EPS = 1e-12
_H = 128
_FIN = 32
_NSEG = 256
_FCH = 128
_ICH = 3


def _pool_matrix() -> np.ndarray:
    P = np.zeros((_H, 4 * _H), np.float32)
    for o in range(_H):
        P[o, 4 * o: 4 * o + 4] = 0.25
    return P


def _resize_taps():
    src = (np.arange(_H, dtype=np.float32) * np.float32(_FIN - 1)) / np.float32(_H - 1)
    i0 = np.floor(src).astype(np.int32)
    i1 = np.minimum(i0 + 1, _FIN - 1)
    w = (src - i0.astype(np.float32)).astype(np.float32)
    return i0, i1, w


def _resize_matrix() -> np.ndarray:
    R = np.zeros((_H, _FIN), np.float32)
    i0, i1, w = _resize_taps()
    for o in range(_H):
        R[o, i0[o]] += np.float32(1.0) - w[o]
        R[o, i1[o]] += w[o]
    return R


_I0, _I1, _W = _resize_taps()
_POOL = _pool_matrix()
_RESIZE = _resize_matrix()

_CT = (((1,), (1,)), ((), ()))
_CS = (((0,), (0,)), ((), ()))


def _sp_kernel(img_ref, f2_ref, ids_ref, p_ref, r_ref, out_ref,
               pooled_ref, g2_ref, m1_ref, m2_ref):
    f32 = jnp.float32
    P = p_ref[...]
    for c in range(_ICH):
        x = jnp.abs(img_ref[0, c])
        t = jnp.dot(P, x, preferred_element_type=f32)
        pc = jax.lax.dot_general(t, P, _CT, preferred_element_type=f32)
        pooled_ref[pl.ds(c * _H, _H), :] = pc

    g2_ref[...] = jax.lax.dot_general(f2_ref[0], r_ref[...], _CT,
                                      preferred_element_type=f32)

    m1_ref[...] = jnp.zeros((8, _NSEG), f32)
    m2_ref[...] = jnp.zeros((_FCH, _NSEG), f32)
    iota_s = jax.lax.broadcasted_iota(jnp.int32, (_NSEG, _H), 0)
    ones_row = jnp.ones((1, _H), f32)
    zeros4 = jnp.zeros((4, _H), f32)

    for pi in range(_H):
        ids_row = ids_ref[0, pi:pi + 1, :]
        oh = (iota_s == ids_row).astype(f32)
        vals = jnp.concatenate(
            [pooled_ref[pi:pi + 1, :],
             pooled_ref[_H + pi:_H + pi + 1, :],
             pooled_ref[2 * _H + pi:2 * _H + pi + 1, :],
             ones_row, zeros4], axis=0)
        m1_ref[...] += jax.lax.dot_general(vals, oh, _CT,
                                           preferred_element_type=f32)
        i0 = int(_I0[pi]); i1 = int(_I1[pi]); w = float(_W[pi])
        g_lo = g2_ref[pl.ds(i0 * _FCH, _FCH), :]
        if w == 0.0 or i1 == i0:
            g_row = g_lo
        else:
            g_row = (1.0 - w) * g_lo + w * g2_ref[pl.ds(i1 * _FCH, _FCH), :]
        m2_ref[...] += jax.lax.dot_general(g_row, oh, _CT,
                                           preferred_element_type=f32)

    cnt = m1_ref[3:4, :]
    inv = jnp.where(cnt > 0, 1.0 / jnp.maximum(cnt, 1.0), 0.0)
    mean1 = m1_ref[0:3, :] * inv
    mean2 = m2_ref[...] * inv

    def pair_norm(m, nch):
        g = jax.lax.dot_general(m, m, _CS, preferred_element_type=f32)
        n = jnp.sum(m * m, axis=0, keepdims=True)
        h = g - n
        d2 = -(h + h.T)
        return jnp.sqrt(jnp.maximum(d2, 0.0) + _EPS) * (1.0 / np.sqrt(nch))

    nrm1 = pair_norm(mean1, _ICH)
    nrm2 = pair_norm(mean2, _FCH)
    s = jnp.sum(jnp.abs(nrm1 - nrm2), keepdims=True)
    s = s * np.float32(1.0 / (_NSEG * _NSEG))
    out_ref[...] = jnp.broadcast_to(s[None], (1, 1, _H))


def _run(image, f2, ids, interpret=False):
    B = image.shape[0]
    return pl.pallas_call(
        _sp_kernel,
        grid=(B,),
        in_specs=[
            pl.BlockSpec((1, _ICH, 512, 512), lambda b: (b, 0, 0, 0)),
            pl.BlockSpec((1, _FCH * _FIN, _FIN), lambda b: (b, 0, 0)),
            pl.BlockSpec((1, _H, _H), lambda b: (b, 0, 0)),
            pl.BlockSpec((_H, 4 * _H), lambda b: (0, 0)),
            pl.BlockSpec((_H, _FIN), lambda b: (0, 0)),
        ],
        out_specs=pl.BlockSpec((1, 1, _H), lambda b: (b, 0, 0)),
        out_shape=jax.ShapeDtypeStruct((B, 1, _H), jnp.float32),
        scratch_shapes=[
            pltpu.VMEM((_ICH * _H, _H), jnp.float32),
            pltpu.VMEM((_FCH * _FIN, _H), jnp.float32),
            pltpu.VMEM((8, _NSEG), jnp.float32),
            pltpu.VMEM((_FCH, _NSEG), jnp.float32),
        ],
        interpret=interpret,
    )(image, f2, ids, jnp.asarray(_POOL), jnp.asarray(_RESIZE))


def kernel(image, feature, sp, num):
    B = image.shape[0]
    ids = jnp.minimum(sp.astype(jnp.int32).reshape(B, _H, _H),
                      jnp.asarray(num, jnp.int32) - 1)
    f2 = feature.transpose(0, 2, 1, 3).reshape(B, _FCH * _FIN, _FIN)
    out = _run(image, f2, ids)
    return jnp.mean(out[:, 0, 0])


# speedup vs baseline: 21.1070x; 1.1870x over previous
import numpy as np

import jax
import jax.numpy as jnp
from jax.experimental import pallas as pl
from jax.experimental.pallas import tpu as pltpu

_EPS = 1e-12
_H = 128
_FIN = 32
_NSEG = 256
_FCH = 128
_ICH = 3


def _pool_matrix() -> np.ndarray:
    P = np.zeros((_H, 4 * _H), np.float32)
    for o in range(_H):
        P[o, 4 * o: 4 * o + 4] = 0.25
    return P


def _resize_taps():
    src = (np.arange(_H, dtype=np.float32) * np.float32(_FIN - 1)) / np.float32(_H - 1)
    i0 = np.floor(src).astype(np.int32)
    i1 = np.minimum(i0 + 1, _FIN - 1)
    w = (src - i0.astype(np.float32)).astype(np.float32)
    return i0, i1, w


def _resize_matrix() -> np.ndarray:
    R = np.zeros((_H, _FIN), np.float32)
    i0, i1, w = _resize_taps()
    for o in range(_H):
        R[o, i0[o]] += np.float32(1.0) - w[o]
        R[o, i1[o]] += w[o]
    return R


_I0, _I1, _W = _resize_taps()
_POOL = _pool_matrix()
_RESIZE = _resize_matrix()

_CT = (((1,), (1,)), ((), ()))
_CS = (((0,), (0,)), ((), ()))


def _sp_kernel(img_ref, f2_ref, ids_ref, p_ref, r_ref, out_ref,
               pooled_ref, g2_ref, m12_ref):
    f32 = jnp.float32
    P = p_ref[...]
    for c in range(_ICH):
        x = jnp.abs(img_ref[0, c])
        t = jnp.dot(P, x, preferred_element_type=f32)
        pc = jax.lax.dot_general(t, P, _CT, preferred_element_type=f32)
        pooled_ref[pl.ds(c * _H, _H), :] = pc

    g2_ref[...] = jax.lax.dot_general(f2_ref[0], r_ref[...], _CT,
                                      preferred_element_type=f32)

    bf16 = jnp.bfloat16
    m12_ref[...] = jnp.zeros((_FCH + 4, _NSEG), f32)
    iota_s = jax.lax.broadcasted_iota(jnp.int32, (_NSEG, 2 * _H), 0).astype(bf16)
    ones_row = jnp.ones((1, _H), bf16)

    def half_lhs(pi):
        i0 = int(_I0[pi]); i1 = int(_I1[pi]); w = float(_W[pi])
        g_lo = g2_ref[pl.ds(i0 * _FCH, _FCH), :]
        if w == 0.0 or i1 == i0:
            g_row = g_lo
        else:
            g_row = (1.0 - w) * g_lo + w * g2_ref[pl.ds(i1 * _FCH, _FCH), :]
        return jnp.concatenate(
            [g_row.astype(bf16),
             pooled_ref[pi:pi + 1, :].astype(bf16),
             pooled_ref[_H + pi:_H + pi + 1, :].astype(bf16),
             pooled_ref[2 * _H + pi:2 * _H + pi + 1, :].astype(bf16),
             ones_row], axis=0)

    for k in range(_H // 2):
        p0, p1 = 2 * k, 2 * k + 1
        ids2 = ids_ref[0, p0:p0 + 2, :].astype(bf16)
        ids_cat = jnp.concatenate([ids2[0:1, :], ids2[1:2, :]], axis=1)
        oh = (iota_s == ids_cat).astype(bf16)
        lhs = jnp.concatenate([half_lhs(p0), half_lhs(p1)], axis=1)
        m12_ref[...] += jax.lax.dot_general(lhs, oh, _CT,
                                            preferred_element_type=f32)

    cnt = m12_ref[_FCH + 3:_FCH + 4, :]
    inv = jnp.where(cnt > 0, 1.0 / jnp.maximum(cnt, 1.0), 0.0)
    mean1 = m12_ref[_FCH:_FCH + 3, :] * inv
    mean2 = m12_ref[0:_FCH, :] * inv

    def pair_norm(m, nch):
        g = jax.lax.dot_general(m, m, _CS, preferred_element_type=f32)
        n = jnp.sum(m * m, axis=0, keepdims=True)
        h = g - n
        d2 = -(h + h.T)
        return jnp.sqrt(jnp.maximum(d2, 0.0) + _EPS) * (1.0 / np.sqrt(nch))

    nrm1 = pair_norm(mean1, _ICH)
    nrm2 = pair_norm(mean2, _FCH)
    s = jnp.sum(jnp.abs(nrm1 - nrm2), keepdims=True)
    s = s * np.float32(1.0 / (_NSEG * _NSEG))
    out_ref[...] = jnp.broadcast_to(s[None], (1, 1, _H))


def _run(image, f2, ids, interpret=False):
    B = image.shape[0]
    return pl.pallas_call(
        _sp_kernel,
        grid=(B,),
        in_specs=[
            pl.BlockSpec((1, _ICH, 512, 512), lambda b: (b, 0, 0, 0)),
            pl.BlockSpec((1, _FCH * _FIN, _FIN), lambda b: (b, 0, 0)),
            pl.BlockSpec((1, _H, _H), lambda b: (b, 0, 0)),
            pl.BlockSpec((_H, 4 * _H), lambda b: (0, 0)),
            pl.BlockSpec((_H, _FIN), lambda b: (0, 0)),
        ],
        out_specs=pl.BlockSpec((1, 1, _H), lambda b: (b, 0, 0)),
        out_shape=jax.ShapeDtypeStruct((B, 1, _H), jnp.float32),
        scratch_shapes=[
            pltpu.VMEM((_ICH * _H, _H), jnp.float32),
            pltpu.VMEM((_FCH * _FIN, _H), jnp.float32),
            pltpu.VMEM((_FCH + 4, _NSEG), jnp.float32),
        ],
        interpret=interpret,
    )(image, f2, ids, jnp.asarray(_POOL), jnp.asarray(_RESIZE))


def kernel(image, feature, sp, num):
    B = image.shape[0]
    ids = jnp.minimum(sp.astype(jnp.int32).reshape(B, _H, _H),
                      jnp.asarray(num, jnp.int32) - 1)
    f2 = feature.transpose(0, 2, 1, 3).reshape(B, _FCH * _FIN, _FIN)
    out = _run(image, f2, ids)
    return jnp.mean(out[:, 0, 0])
